# baseline (speedup 1.0000x reference)
import functools

import jax
import jax.numpy as jnp
from jax import lax
from jax.experimental import pallas as pl
from jax.experimental.pallas import tpu as pltpu
from jax.experimental.pallas import tpu_sc as plsc

BLK = 512
H_TILE = 512
SC_CHUNK = 32


def _ffn_body(nk, nb_max, s_exp, s_nbt,
              x_ref, w1_ref, b1_ref, w2_ref, b2_ref, y_hbm,
              ys_v, sem_out):
    k = pl.program_id(0)
    b_raw = pl.program_id(1)
    b = jnp.where(k % 2 == 1, nb_max - 1 - b_raw, b_raw)

    @pl.when(b < s_nbt[0])
    def _():
        xb = x_ref[...]
        h = lax.dot_general(xb, w1_ref[0], (((1,), (1,)), ((), ())),
                            preferred_element_type=jnp.float32)
        h = jnp.maximum(h + b1_ref[0, 0], 0.0)
        yp = lax.dot_general(h, w2_ref[0], (((1,), (1,)), ((), ())),
                             preferred_element_type=jnp.float32)

        @pl.when(k == 0)
        def _():
            ys_v[pl.ds(b * BLK, BLK), :] = yp + b2_ref[0]

        @pl.when(k != 0)
        def _():
            ys_v[pl.ds(b * BLK, BLK), :] += yp

    @pl.when((k == nk - 1) & (b_raw == nb_max - 1))
    def _():
        pltpu.make_async_copy(ys_v, y_hbm, sem_out).start()
        pltpu.make_async_copy(ys_v, y_hbm, sem_out).wait()


def _ffn(x_sorted, W1, b1, W2, b2, s_exp, s_nbt, nb_max):
    E, H, D = W1.shape
    nk = H // H_TILE
    npad = x_sorted.shape[0]

    def _bb(k, b):
        return jnp.where(k % 2 == 1, nb_max - 1 - b, b)

    grid_spec = pltpu.PrefetchScalarGridSpec(
        num_scalar_prefetch=2,
        grid=(nk, nb_max),
        in_specs=[
            pl.BlockSpec((BLK, D), lambda k, b, se, sn: (_bb(k, b), 0)),
            pl.BlockSpec((1, H_TILE, D), lambda k, b, se, sn: (se[_bb(k, b)], k, 0)),
            pl.BlockSpec((1, 1, 1, H_TILE), lambda k, b, se, sn: (se[_bb(k, b)], k, 0, 0)),
            pl.BlockSpec((1, D, H_TILE), lambda k, b, se, sn: (se[_bb(k, b)], 0, k)),
            pl.BlockSpec((1, 1, D), lambda k, b, se, sn: (se[_bb(k, b)], 0, 0)),
        ],
        out_specs=pl.BlockSpec(memory_space=pl.ANY),
        scratch_shapes=[
            pltpu.VMEM((npad, D), jnp.float32),
            pltpu.SemaphoreType.DMA,
        ],
    )
    return pl.pallas_call(
        functools.partial(_ffn_body, nk, nb_max),
        grid_spec=grid_spec,
        out_shape=jax.ShapeDtypeStruct((npad, D), jnp.float32),
        compiler_params=pltpu.CompilerParams(
            dimension_semantics=("arbitrary", "arbitrary")),
    )(s_exp, s_nbt, x_sorted, W1, b1.reshape(E, nk, 1, H_TILE),
      W2, b2.reshape(E, 1, D))


def _sc_gather_rows(table, idx):
    n = idx.shape[0]
    d = table.shape[1]
    info = plsc.get_sparse_core_info()
    nw = info.num_cores * info.num_subcores
    rpw = n // nw
    nch = rpw // SC_CHUNK
    mesh = plsc.VectorSubcoreMesh(core_axis_name="c", subcore_axis_name="s")

    @functools.partial(
        pl.kernel,
        mesh=mesh,
        out_type=jax.ShapeDtypeStruct((n, d), table.dtype),
        scratch_types=[
            pltpu.VMEM((rpw,), jnp.int32),
            pltpu.VMEM((SC_CHUNK, d), table.dtype),
            pltpu.SemaphoreType.DMA,
        ],
    )
    def k(table_hbm, idx_hbm, out_hbm, idx_v, rows_v, sem):
        wid = lax.axis_index("s") * info.num_cores + lax.axis_index("c")
        base = wid * rpw
        pltpu.sync_copy(idx_hbm.at[pl.ds(base, rpw)], idx_v)
        for c in range(nch):
            pltpu.async_copy(
                table_hbm.at[idx_v.at[pl.ds(c * SC_CHUNK, SC_CHUNK)]],
                rows_v, sem).wait()
            pltpu.sync_copy(rows_v, out_hbm.at[pl.ds(base + c * SC_CHUNK, SC_CHUNK)])

    return k(table, idx)


def _routing(top1, T, E, nb_max):
    onehot = (top1[:, None] == jnp.arange(E, dtype=jnp.int32)[None, :])
    csum = jnp.cumsum(onehot.astype(jnp.int32), axis=0)
    counts = csum[-1]
    rank = jnp.take_along_axis(csum, top1[:, None], axis=1)[:, 0] - 1
    nb = (counts + BLK - 1) // BLK
    nb_cum = jnp.cumsum(nb).astype(jnp.int32)
    nb_off = nb_cum - nb
    nb_total = nb_cum[-1]

    bids = jnp.arange(nb_max, dtype=jnp.int32)
    e_of_b = jnp.minimum(
        jnp.searchsorted(nb_cum, bids, side="right").astype(jnp.int32), E - 1)
    s_exp = e_of_b
    s_nbt = nb_total.reshape(1)

    posp = nb_off[top1] * BLK + rank
    gather_idx = jnp.zeros((nb_max * BLK,), jnp.int32).at[posp].set(
        jnp.arange(T, dtype=jnp.int32))
    return s_exp, s_nbt, gather_idx, posp


def kernel(x, gate_W, gate_b, W1, b1, W2, b2):
    Bn, Sn, D = x.shape
    T = Bn * Sn
    E, H, _ = W1.shape
    nb_max = T // BLK + E

    x_flat = x.reshape(T, D)
    gate_logits = x_flat @ gate_W.T + gate_b
    top1 = jnp.argmax(gate_logits, axis=-1).astype(jnp.int32)

    s_exp, s_nbt, gather_idx, inv = _routing(top1, T, E, nb_max)

    x_sorted = _sc_gather_rows(x_flat, gather_idx)
    y_sorted = _ffn(x_sorted, W1, b1, W2, b2, s_exp, s_nbt, nb_max)
    out_flat = _sc_gather_rows(y_sorted, inv)
    return out_flat.reshape(Bn, Sn, D)


# speedup vs baseline: 1.4334x; 1.4334x over previous
import functools

import jax
import jax.numpy as jnp
from jax import lax
from jax.experimental import pallas as pl
from jax.experimental.pallas import tpu as pltpu
from jax.experimental.pallas import tpu_sc as plsc

BLK = 256
H_TILE = 512
SC_CHUNK = 32


def _ffn_body(nk, ne, s_nb, s_noff,
              x_hbm, w1_ref, b1_ref, w2_ref, b2_ref, y_hbm,
              xs_v, ys_v, sem_in, sem_out):
    k = pl.program_id(0)
    e_raw = pl.program_id(1)
    e = jnp.where(k % 2 == 1, ne - 1 - e_raw, e_raw)
    nblk = s_nb[e]
    roff = s_noff[e] * BLK

    @pl.when((k == 0) & (e_raw == 0))
    def _():
        pltpu.make_async_copy(x_hbm, xs_v, sem_in).start()
        pltpu.make_async_copy(x_hbm, xs_v, sem_in).wait()

    def _block(i, base):
        xb = xs_v[pl.ds(base + i * BLK, BLK), :]
        h = lax.dot_general(xb, w1_ref[0], (((1,), (1,)), ((), ())),
                            preferred_element_type=jnp.float32)
        h = jnp.maximum(h + b1_ref[0, 0], 0.0)
        return lax.dot_general(h, w2_ref[0], (((1,), (1,)), ((), ())),
                               preferred_element_type=jnp.float32)

    @pl.when(k == 0)
    def _():
        def body(i, c):
            ys_v[pl.ds(roff + i * BLK, BLK), :] = _block(i, roff) + b2_ref[0]
            return c
        lax.fori_loop(0, nblk, body, 0)

    @pl.when(k != 0)
    def _():
        def body(i, c):
            ys_v[pl.ds(roff + i * BLK, BLK), :] += _block(i, roff)
            return c
        lax.fori_loop(0, nblk, body, 0)

    @pl.when((k == nk - 1) & (e_raw == ne - 1))
    def _():
        pltpu.make_async_copy(ys_v, y_hbm, sem_out).start()
        pltpu.make_async_copy(ys_v, y_hbm, sem_out).wait()


def _ffn(x_sorted, W1, b1, W2, b2, s_nb, s_noff):
    E, H, D = W1.shape
    nk = H // H_TILE
    npad = x_sorted.shape[0]

    def _ee(k, e):
        return jnp.where(k % 2 == 1, E - 1 - e, e)

    grid_spec = pltpu.PrefetchScalarGridSpec(
        num_scalar_prefetch=2,
        grid=(nk, E),
        in_specs=[
            pl.BlockSpec(memory_space=pl.ANY),
            pl.BlockSpec((1, H_TILE, D), lambda k, e, snb, sno: (_ee(k, e), k, 0)),
            pl.BlockSpec((1, 1, 1, H_TILE), lambda k, e, snb, sno: (_ee(k, e), k, 0, 0)),
            pl.BlockSpec((1, D, H_TILE), lambda k, e, snb, sno: (_ee(k, e), 0, k)),
            pl.BlockSpec((1, 1, D), lambda k, e, snb, sno: (_ee(k, e), 0, 0)),
        ],
        out_specs=pl.BlockSpec(memory_space=pl.ANY),
        scratch_shapes=[
            pltpu.VMEM((npad, D), jnp.float32),
            pltpu.VMEM((npad, D), jnp.float32),
            pltpu.SemaphoreType.DMA,
            pltpu.SemaphoreType.DMA,
        ],
    )
    return pl.pallas_call(
        functools.partial(_ffn_body, nk, E),
        grid_spec=grid_spec,
        out_shape=jax.ShapeDtypeStruct((npad, D), jnp.float32),
        compiler_params=pltpu.CompilerParams(
            dimension_semantics=("arbitrary", "arbitrary")),
    )(s_nb, s_noff, x_sorted, W1, b1.reshape(E, nk, 1, H_TILE),
      W2, b2.reshape(E, 1, D))


def _sc_gather_rows(table, idx):
    n = idx.shape[0]
    d = table.shape[1]
    info = plsc.get_sparse_core_info()
    nw = info.num_cores * info.num_subcores
    rpw = n // nw
    nch = rpw // SC_CHUNK
    mesh = plsc.VectorSubcoreMesh(core_axis_name="c", subcore_axis_name="s")

    @functools.partial(
        pl.kernel,
        mesh=mesh,
        out_type=jax.ShapeDtypeStruct((n, d), table.dtype),
        scratch_types=[
            pltpu.VMEM((rpw,), jnp.int32),
            pltpu.VMEM((SC_CHUNK, d), table.dtype),
            pltpu.SemaphoreType.DMA,
        ],
    )
    def k(table_hbm, idx_hbm, out_hbm, idx_v, rows_v, sem):
        wid = lax.axis_index("s") * info.num_cores + lax.axis_index("c")
        base = wid * rpw
        pltpu.sync_copy(idx_hbm.at[pl.ds(base, rpw)], idx_v)
        for c in range(nch):
            pltpu.async_copy(
                table_hbm.at[idx_v.at[pl.ds(c * SC_CHUNK, SC_CHUNK)]],
                rows_v, sem).wait()
            pltpu.sync_copy(rows_v, out_hbm.at[pl.ds(base + c * SC_CHUNK, SC_CHUNK)])

    return k(table, idx)


def _routing(top1, T, E, nb_max):
    onehot = (top1[:, None] == jnp.arange(E, dtype=jnp.int32)[None, :])
    csum = jnp.cumsum(onehot.astype(jnp.int32), axis=0)
    counts = csum[-1]
    rank = jnp.take_along_axis(csum, top1[:, None], axis=1)[:, 0] - 1
    nb = (counts + BLK - 1) // BLK
    nb_cum = jnp.cumsum(nb).astype(jnp.int32)
    nb_off = nb_cum - nb
    nb_total = nb_cum[-1]

    posp = nb_off[top1] * BLK + rank
    gather_idx = jnp.zeros((nb_max * BLK,), jnp.int32).at[posp].set(
        jnp.arange(T, dtype=jnp.int32))
    return nb.astype(jnp.int32), nb_off.astype(jnp.int32), gather_idx, posp


def kernel(x, gate_W, gate_b, W1, b1, W2, b2):
    Bn, Sn, D = x.shape
    T = Bn * Sn
    E, H, _ = W1.shape
    nb_max = T // BLK + E

    x_flat = x.reshape(T, D)
    gate_logits = x_flat @ gate_W.T + gate_b
    top1 = jnp.argmax(gate_logits, axis=-1).astype(jnp.int32)

    s_nb, s_noff, gather_idx, inv = _routing(top1, T, E, nb_max)

    x_sorted = _sc_gather_rows(x_flat, gather_idx)
    y_sorted = _ffn(x_sorted, W1, b1, W2, b2, s_nb, s_noff)
    out_flat = _sc_gather_rows(y_sorted, inv)
    return out_flat.reshape(Bn, Sn, D)


# speedup vs baseline: 2.2237x; 1.5514x over previous
import functools

import jax
import jax.numpy as jnp
from jax import lax
from jax.experimental import pallas as pl
from jax.experimental.pallas import tpu as pltpu
from jax.experimental.pallas import tpu_sc as plsc

BLK = 256
H_TILE = 512
SC_CHUNK = 32


def _ffn_body(nk, ne, s_nb, s_noff,
              x_hbm, w1_ref, b1_ref, w2_ref, b2_ref, y_hbm,
              xs_v, ys_v, sem_in, sem_out):
    k = pl.program_id(0)
    e_raw = pl.program_id(1)
    e = jnp.where(k % 2 == 1, ne - 1 - e_raw, e_raw)
    nblk = s_nb[e]
    roff = s_noff[e] * BLK

    @pl.when((k == 0) & (e_raw == 0))
    def _():
        pltpu.make_async_copy(x_hbm, xs_v, sem_in).start()
        pltpu.make_async_copy(x_hbm, xs_v, sem_in).wait()

    def _block(i, base):
        xb = xs_v[pl.ds(base + i * BLK, BLK), :]
        h = lax.dot_general(xb, w1_ref[0], (((1,), (1,)), ((), ())),
                            preferred_element_type=jnp.float32)
        h = jnp.maximum(h + b1_ref[0, 0], 0.0)
        return lax.dot_general(h, w2_ref[0], (((1,), (1,)), ((), ())),
                               preferred_element_type=jnp.float32)

    @pl.when(k == 0)
    def _():
        def body(i, c):
            ys_v[pl.ds(roff + i * BLK, BLK), :] = _block(i, roff) + b2_ref[0]
            return c
        lax.fori_loop(0, nblk, body, 0)

    @pl.when(k != 0)
    def _():
        def body(i, c):
            ys_v[pl.ds(roff + i * BLK, BLK), :] += _block(i, roff)
            return c
        lax.fori_loop(0, nblk, body, 0)

    @pl.when((k == nk - 1) & (e_raw == ne - 1))
    def _():
        pltpu.make_async_copy(ys_v, y_hbm, sem_out).start()
        pltpu.make_async_copy(ys_v, y_hbm, sem_out).wait()


def _ffn(x_sorted, W1, b1, W2, b2, s_nb, s_noff):
    E, H, D = W1.shape
    nk = H // H_TILE
    npad = x_sorted.shape[0]

    def _ee(k, e):
        return jnp.where(k % 2 == 1, E - 1 - e, e)

    grid_spec = pltpu.PrefetchScalarGridSpec(
        num_scalar_prefetch=2,
        grid=(nk, E),
        in_specs=[
            pl.BlockSpec(memory_space=pl.ANY),
            pl.BlockSpec((1, H_TILE, D), lambda k, e, snb, sno: (_ee(k, e), k, 0)),
            pl.BlockSpec((1, 1, 1, H_TILE), lambda k, e, snb, sno: (_ee(k, e), k, 0, 0)),
            pl.BlockSpec((1, D, H_TILE), lambda k, e, snb, sno: (_ee(k, e), 0, k)),
            pl.BlockSpec((1, 1, D), lambda k, e, snb, sno: (_ee(k, e), 0, 0)),
        ],
        out_specs=pl.BlockSpec(memory_space=pl.ANY),
        scratch_shapes=[
            pltpu.VMEM((npad, D), jnp.float32),
            pltpu.VMEM((npad, D), jnp.float32),
            pltpu.SemaphoreType.DMA,
            pltpu.SemaphoreType.DMA,
        ],
    )
    return pl.pallas_call(
        functools.partial(_ffn_body, nk, E),
        grid_spec=grid_spec,
        out_shape=jax.ShapeDtypeStruct((npad, D), jnp.float32),
        compiler_params=pltpu.CompilerParams(
            dimension_semantics=("arbitrary", "arbitrary")),
    )(s_nb, s_noff, x_sorted, W1, b1.reshape(E, nk, 1, H_TILE),
      W2, b2.reshape(E, 1, D))


def _sc_gather_rows(table, idx):
    n = idx.shape[0]
    d = table.shape[1]
    info = plsc.get_sparse_core_info()
    nw = info.num_cores * info.num_subcores
    rpw = n // nw
    nch = rpw // SC_CHUNK
    mesh = plsc.VectorSubcoreMesh(core_axis_name="c", subcore_axis_name="s")

    @functools.partial(
        pl.kernel,
        mesh=mesh,
        out_type=jax.ShapeDtypeStruct((n, d), table.dtype),
        scratch_types=[
            pltpu.VMEM((rpw,), jnp.int32),
            pltpu.VMEM((SC_CHUNK, d), table.dtype),
            pltpu.SemaphoreType.DMA,
        ],
    )
    def k(table_hbm, idx_hbm, out_hbm, idx_v, rows_v, sem):
        wid = lax.axis_index("s") * info.num_cores + lax.axis_index("c")
        base = wid * rpw
        pltpu.sync_copy(idx_hbm.at[pl.ds(base, rpw)], idx_v)
        for c in range(nch):
            pltpu.async_copy(
                table_hbm.at[idx_v.at[pl.ds(c * SC_CHUNK, SC_CHUNK)]],
                rows_v, sem).wait()
            pltpu.sync_copy(rows_v, out_hbm.at[pl.ds(base + c * SC_CHUNK, SC_CHUNK)])

    return k(table, idx)


def _sc_scatter_rows(rows, idx3, npad):
    t, d = rows.shape
    nw, nch, _ = idx3.shape
    rpw = t // nw
    mesh = plsc.VectorSubcoreMesh(core_axis_name="c", subcore_axis_name="s")

    @functools.partial(
        pl.kernel,
        mesh=mesh,
        out_type=jax.ShapeDtypeStruct((npad, d), rows.dtype),
        scratch_types=[
            pltpu.VMEM((nch, SC_CHUNK), jnp.int32),
            pltpu.VMEM((SC_CHUNK, d), rows.dtype),
            pltpu.SemaphoreType.DMA,
        ],
    )
    def k(rows_hbm, idx_hbm, out_hbm, idx_v, buf_v, sem):
        info = plsc.get_sparse_core_info()
        wid = lax.axis_index("s") * info.num_cores + lax.axis_index("c")
        base = wid * rpw
        pltpu.sync_copy(idx_hbm.at[wid], idx_v)
        for c in range(nch):
            pltpu.sync_copy(rows_hbm.at[pl.ds(base + c * SC_CHUNK, SC_CHUNK)], buf_v)
            pltpu.async_copy(buf_v, out_hbm.at[idx_v.at[c]], sem).wait()

    return k(rows, idx3)


def _route_body(ne, top_ref, posp_ref, nb_ref, nboff_ref):
    top = top_ref[...]
    R, C = top.shape
    U = (lax.broadcasted_iota(jnp.int32, (C, C), 0)
         <= lax.broadcasted_iota(jnp.int32, (C, C), 1)).astype(jnp.float32)
    S = (lax.broadcasted_iota(jnp.int32, (R, R), 1)
         < lax.broadcasted_iota(jnp.int32, (R, R), 0)).astype(jnp.float32)
    ones_c = jnp.ones((C, C), jnp.float32)

    posp = jnp.zeros((R, C), jnp.int32)
    nb_vec = jnp.zeros((1, ne), jnp.int32)
    nboff_vec = jnp.zeros((1, ne), jnp.int32)
    eids = lax.broadcasted_iota(jnp.int32, (1, ne), 1)
    nboff = jnp.int32(0)
    for e in range(ne):
        m = (top == e).astype(jnp.float32)
        incl = lax.dot_general(m, U, (((1,), (0,)), ((), ())),
                               preferred_element_type=jnp.float32)
        rs = lax.dot_general(m, ones_c, (((1,), (0,)), ((), ())),
                             preferred_element_type=jnp.float32)
        pre = lax.dot_general(S, rs, (((1,), (0,)), ((), ())),
                              preferred_element_type=jnp.float32)
        rank = (incl + pre).astype(jnp.int32) - 1
        cnt = (pre + rs)[R - 1, 0].astype(jnp.int32)
        nb_e = (cnt + BLK - 1) // BLK
        posp = jnp.where(top == e, nboff * BLK + rank, posp)
        oh = (eids == e).astype(jnp.int32)
        nb_vec = nb_vec + oh * nb_e
        nboff_vec = nboff_vec + oh * nboff
        nboff = nboff + nb_e
    posp_ref[...] = posp
    nb_ref[...] = nb_vec
    nboff_ref[...] = nboff_vec


def _routing(top1, T, E, nb_max):
    R = 32
    C = T // R
    posp2, nbv, nboffv = pl.pallas_call(
        functools.partial(_route_body, E),
        out_shape=(jax.ShapeDtypeStruct((R, C), jnp.int32),
                   jax.ShapeDtypeStruct((1, E), jnp.int32),
                   jax.ShapeDtypeStruct((1, E), jnp.int32)),
    )(top1.reshape(R, C))
    return nbv.reshape(E), nboffv.reshape(E), posp2.reshape(T)


def kernel(x, gate_W, gate_b, W1, b1, W2, b2):
    Bn, Sn, D = x.shape
    T = Bn * Sn
    E, H, _ = W1.shape
    nb_max = T // BLK + E

    x_flat = x.reshape(T, D)
    gate_logits = x_flat @ gate_W.T + gate_b
    top1 = jnp.argmax(gate_logits, axis=-1).astype(jnp.int32)

    s_nb, s_noff, posp = _routing(top1, T, E, nb_max)

    nw = 32
    posp3 = posp.reshape(nw, (T // nw) // SC_CHUNK, SC_CHUNK)
    x_sorted = _sc_scatter_rows(x_flat, posp3, nb_max * BLK)
    y_sorted = _ffn(x_sorted, W1, b1, W2, b2, s_nb, s_noff)
    out_flat = _sc_gather_rows(y_sorted, posp)
    return out_flat.reshape(Bn, Sn, D)
